# baseline (speedup 1.0000x reference)
import functools

import jax
import jax.numpy as jnp
from jax import lax
from jax.experimental import pallas as pl
from jax.experimental.pallas import tpu as pltpu
from jax.experimental.pallas import tpu_sc as plsc

_VOCAB = 100000
_DIM = 128
_BATCH = 4096
_NC = 2
_NS = 16
_L = 16
_NW = _NC * _NS
_BPW = _BATCH // _NW
_CHUNK = 64
_NCH = _BPW // _CHUNK


def _sc_partials(word, context, U, V):
    mesh = plsc.VectorSubcoreMesh(core_axis_name="c", subcore_axis_name="s")

    @functools.partial(
        pl.kernel,
        mesh=mesh,
        out_type=jax.ShapeDtypeStruct((_NW, _L), jnp.float32),
        scratch_types=[
            pltpu.VMEM((_BPW,), jnp.int32),
            pltpu.VMEM((_BPW,), jnp.int32),
            pltpu.VMEM((_BPW, _DIM), jnp.float32),
            pltpu.VMEM((_BPW, _DIM), jnp.float32),
            pltpu.VMEM((_L,), jnp.float32),
        ] + [pltpu.SemaphoreType.DMA] * (2 * _NCH),
    )
    def k(word_hbm, ctx_hbm, u_hbm, v_hbm, out_hbm,
          widx, cidx, urows, vrows, sres, *sems):
        wid = lax.axis_index("s") * _NC + lax.axis_index("c")
        base = wid * _BPW
        pltpu.sync_copy(word_hbm.at[pl.ds(base, _BPW)], widx)
        pltpu.sync_copy(ctx_hbm.at[pl.ds(base, _BPW)], cidx)
        copies = []
        for ch in range(_NCH):
            sl = pl.ds(ch * _CHUNK, _CHUNK)
            copies.append(
                pltpu.async_copy(u_hbm.at[widx.at[sl]], urows.at[sl],
                                 sems[2 * ch]))
            copies.append(
                pltpu.async_copy(v_hbm.at[cidx.at[sl]], vrows.at[sl],
                                 sems[2 * ch + 1]))

        nacc = _DIM // _L

        def row(i, accs):
            return tuple(
                accs[j] + (urows[i, pl.ds(j * _L, _L)]
                           * vrows[i, pl.ds(j * _L, _L)])
                for j in range(nacc))

        accs = tuple(jnp.zeros((_L,), jnp.float32) for _ in range(nacc))
        for ch in range(_NCH):
            copies[2 * ch].wait()
            copies[2 * ch + 1].wait()
            accs = lax.fori_loop(ch * _CHUNK, (ch + 1) * _CHUNK, row, accs)
        acc = accs[0]
        for j in range(1, nacc):
            acc = acc + accs[j]
        sres[...] = acc
        pltpu.sync_copy(sres, out_hbm.at[wid])

    return k(word, context, U, V)


def _finalize(partials):
    def body(p_ref, o_ref):
        s = jnp.sum(p_ref[...])
        o_ref[...] = jnp.broadcast_to(jax.nn.log_sigmoid(s), (1, 1))

    return pl.pallas_call(
        body,
        out_shape=jax.ShapeDtypeStruct((1, 1), jnp.float32),
    )(partials)


def kernel(word, context, U, V):
    partials = _sc_partials(word.astype(jnp.int32), context.astype(jnp.int32),
                            U, V)
    return _finalize(partials)


# speedup vs baseline: 1.0087x; 1.0087x over previous
import functools

import jax
import jax.numpy as jnp
from jax import lax
from jax.experimental import pallas as pl
from jax.experimental.pallas import tpu as pltpu
from jax.experimental.pallas import tpu_sc as plsc

_VOCAB = 100000
_DIM = 128
_BATCH = 4096
_NC = 2
_NS = 16
_L = 16
_NW = _NC * _NS
_BPW = _BATCH // _NW
_CHUNK = 128
_NCH = _BPW // _CHUNK


def _sc_partials(word, context, U, V):
    mesh = plsc.VectorSubcoreMesh(core_axis_name="c", subcore_axis_name="s")

    @functools.partial(
        pl.kernel,
        mesh=mesh,
        out_type=jax.ShapeDtypeStruct((_NW, _L), jnp.float32),
        scratch_types=[
            pltpu.VMEM((_BPW,), jnp.int32),
            pltpu.VMEM((_BPW,), jnp.int32),
            pltpu.VMEM((_BPW, _DIM), jnp.float32),
            pltpu.VMEM((_BPW, _DIM), jnp.float32),
            pltpu.VMEM((_L,), jnp.float32),
        ] + [pltpu.SemaphoreType.DMA] * (2 * _NCH),
    )
    def k(word_hbm, ctx_hbm, u_hbm, v_hbm, out_hbm,
          widx, cidx, urows, vrows, sres, *sems):
        wid = lax.axis_index("s") * _NC + lax.axis_index("c")
        base = wid * _BPW
        pltpu.sync_copy(word_hbm.at[pl.ds(base, _BPW)], widx)
        pltpu.sync_copy(ctx_hbm.at[pl.ds(base, _BPW)], cidx)
        copies = []
        for ch in range(_NCH):
            sl = pl.ds(ch * _CHUNK, _CHUNK)
            copies.append(
                pltpu.async_copy(u_hbm.at[widx.at[sl]], urows.at[sl],
                                 sems[2 * ch]))
            copies.append(
                pltpu.async_copy(v_hbm.at[cidx.at[sl]], vrows.at[sl],
                                 sems[2 * ch + 1]))

        nacc = _DIM // _L

        def row(i, accs):
            return tuple(
                accs[j] + (urows[i, pl.ds(j * _L, _L)]
                           * vrows[i, pl.ds(j * _L, _L)])
                for j in range(nacc))

        accs = tuple(jnp.zeros((_L,), jnp.float32) for _ in range(nacc))
        for ch in range(_NCH):
            copies[2 * ch].wait()
            copies[2 * ch + 1].wait()
            accs = lax.fori_loop(ch * _CHUNK, (ch + 1) * _CHUNK, row, accs)
        acc = accs[0]
        for j in range(1, nacc):
            acc = acc + accs[j]
        sres[...] = acc
        pltpu.sync_copy(sres, out_hbm.at[wid])

    return k(word, context, U, V)


def _finalize(partials):
    def body(p_ref, o_ref):
        s = jnp.sum(p_ref[...])
        o_ref[...] = jnp.broadcast_to(jax.nn.log_sigmoid(s), (1, 1))

    return pl.pallas_call(
        body,
        out_shape=jax.ShapeDtypeStruct((1, 1), jnp.float32),
    )(partials)


def kernel(word, context, U, V):
    partials = _sc_partials(word.astype(jnp.int32), context.astype(jnp.int32),
                            U, V)
    return _finalize(partials)
